# baseline (speedup 1.0000x reference)
import functools

import jax
import jax.numpy as jnp
from jax import lax
from jax.experimental import pallas as pl
from jax.experimental.pallas import tpu as pltpu
from jax.experimental.pallas import tpu_sc as plsc

VOCAB = 50257
T = 2048
D = 768
B = 4
NC, NS, L = 2, 16, 16
NW = NC * NS
T_PER_W = T // NW
C = 32
NCH = T_PER_W // C
VPR = D // L


def _body(idx_hbm, tok_hbm, pos_hbm, out_hbm, idx_v, pos_v, rows_v, sem):
    cid = lax.axis_index("c")
    sid = lax.axis_index("s")
    wid = sid * NC + cid
    t0 = wid * T_PER_W

    for b in range(B):
        for g in range(NCH):
            pltpu.sync_copy(idx_hbm.at[b, pl.ds(t0 + g * C, C)],
                            idx_v.at[b * NCH + g])

    for g in range(NCH):
        pltpu.sync_copy(pos_hbm.at[pl.ds(t0 + g * C, C)], pos_v)
        for b in range(B):
            pltpu.async_copy(tok_hbm.at[idx_v.at[b * NCH + g]], rows_v, sem).wait()

            def _row(r, _):
                def _vec(j, _):
                    v = pos_v[r, pl.ds(j * L, L)]
                    plsc.addupdate(rows_v.at[r, pl.ds(j * L, L)], v)
                    return 0
                lax.fori_loop(0, VPR, _vec, 0, unroll=4)
                return 0
            lax.fori_loop(0, C, _row, 0)

            pltpu.sync_copy(rows_v, out_hbm.at[pl.ds(b * T + t0 + g * C, C)])


@jax.jit
def _run(idx, tok_emb, pos_emb):
    k = pl.kernel(
        _body,
        out_type=jax.ShapeDtypeStruct((B * T, D), jnp.float32),
        mesh=plsc.VectorSubcoreMesh(core_axis_name="c", subcore_axis_name="s"),
        scratch_types=[
            pltpu.VMEM((B * NCH, C), jnp.int32),
            pltpu.VMEM((C, D), jnp.float32),
            pltpu.VMEM((C, D), jnp.float32),
            pltpu.SemaphoreType.DMA,
        ],
    )
    return k(idx, tok_emb, pos_emb)


def kernel(idx, tok_emb, pos_emb):
    out = _run(idx.astype(jnp.int32), tok_emb, pos_emb)
    return out.reshape(B, T, D)


# speedup vs baseline: 1.7353x; 1.7353x over previous
import jax
import jax.numpy as jnp
from jax import lax
from jax.experimental import pallas as pl
from jax.experimental.pallas import tpu as pltpu
from jax.experimental.pallas import tpu_sc as plsc

T = 2048
D = 768
B = 4
NC, NS, L = 2, 16, 16
NW = NC * NS
T_PER_W = T // NW
C = 16
NCH = T_PER_W // C
VPR = D // L


def _body(idx_hbm, tok_hbm, pos_hbm, out_hbm, idx_v, pos_v, rows_v, gsem, psem, ssem):
    cid = lax.axis_index("c")
    sid = lax.axis_index("s")
    wid = sid * NC + cid
    t0 = wid * T_PER_W

    for b in range(B):
        for g in range(NCH):
            pltpu.sync_copy(idx_hbm.at[b, pl.ds(t0 + g * C, C)],
                            idx_v.at[b * NCH + g])

    def fire(g):
        s = g % 2
        p = pltpu.async_copy(pos_hbm.at[pl.ds(t0 + g * C, C)], pos_v.at[s], psem)
        gs = [pltpu.async_copy(tok_hbm.at[idx_v.at[b * NCH + g]],
                               rows_v.at[s, b], gsem) for b in range(B)]
        return [p] + gs

    inflight = {0: fire(0)}
    stores = {}
    for g in range(NCH):
        s = g % 2
        if g + 1 < NCH:
            if g >= 1:
                for d in stores.pop(g - 1):
                    d.wait()
            inflight[g + 1] = fire(g + 1)
        for d in inflight.pop(g):
            d.wait()

        def _row(r, _):
            def _vec(j, _):
                v = pos_v[s, r, pl.ds(j * L, L)]
                for b in range(B):
                    plsc.addupdate(rows_v.at[s, b, r, pl.ds(j * L, L)], v)
                return 0
            lax.fori_loop(0, VPR, _vec, 0, unroll=4)
            return 0
        lax.fori_loop(0, C, _row, 0)

        stores[g] = [pltpu.async_copy(rows_v.at[s, b],
                                      out_hbm.at[pl.ds(b * T + t0 + g * C, C)], ssem)
                     for b in range(B)]
    for g in stores:
        for d in stores[g]:
            d.wait()


@jax.jit
def _run(idx, tok_emb, pos_emb):
    k = pl.kernel(
        _body,
        out_type=jax.ShapeDtypeStruct((B * T, D), jnp.float32),
        mesh=plsc.VectorSubcoreMesh(core_axis_name="c", subcore_axis_name="s"),
        scratch_types=[
            pltpu.VMEM((B * NCH, C), jnp.int32),
            pltpu.VMEM((2, C, D), jnp.float32),
            pltpu.VMEM((2, B, C, D), jnp.float32),
            pltpu.SemaphoreType.DMA,
            pltpu.SemaphoreType.DMA,
            pltpu.SemaphoreType.DMA,
        ],
    )
    return k(idx, tok_emb, pos_emb)


def kernel(idx, tok_emb, pos_emb):
    out = _run(idx.astype(jnp.int32), tok_emb, pos_emb)
    return out.reshape(B, T, D)


# speedup vs baseline: 1.9789x; 1.1404x over previous
import jax
import jax.numpy as jnp
from jax import lax
from jax.experimental import pallas as pl
from jax.experimental.pallas import tpu as pltpu
from jax.experimental.pallas import tpu_sc as plsc

T = 2048
D = 768
B = 4
NC, NS, L = 2, 16, 16
NW = NC * NS
T_PER_W = T // NW
C = 16
NCH = T_PER_W // C
VPR = D // L


def _body(idx_hbm, tok_hbm, pos_hbm, out_hbm, idx_v, pos_v, rows_v, gsem, psem, ssem):
    cid = lax.axis_index("c")
    sid = lax.axis_index("s")
    wid = sid * NC + cid
    t0 = wid * T_PER_W

    idx_cps = [pltpu.async_copy(idx_hbm.at[b, pl.ds(t0, T_PER_W)],
                                idx_v.at[b], psem) for b in range(B)]
    for d in idx_cps:
        d.wait()

    def fire(g):
        s = g % 2
        p = pltpu.async_copy(pos_hbm.at[pl.ds(t0 + g * C, C)], pos_v.at[s], psem)
        gs = [pltpu.async_copy(tok_hbm.at[idx_v.at[b, pl.ds(g * C, C)]],
                               rows_v.at[s, b], gsem) for b in range(B)]
        return [p] + gs

    inflight = {0: fire(0)}
    stores = {}
    for g in range(NCH):
        s = g % 2
        if g + 1 < NCH:
            if g >= 1:
                for d in stores.pop(g - 1):
                    d.wait()
            inflight[g + 1] = fire(g + 1)
        for d in inflight.pop(g):
            d.wait()

        def _row(r, _):
            def _vec(j, _):
                v = pos_v[s, r, pl.ds(j * L, L)]
                for b in range(B):
                    plsc.addupdate(rows_v.at[s, b, r, pl.ds(j * L, L)], v)
                return 0
            lax.fori_loop(0, VPR, _vec, 0, unroll=4)
            return 0
        lax.fori_loop(0, C, _row, 0)

        stores[g] = [pltpu.async_copy(rows_v.at[s, b],
                                      out_hbm.at[pl.ds(b * T + t0 + g * C, C)], ssem)
                     for b in range(B)]
    for g in stores:
        for d in stores[g]:
            d.wait()


@jax.jit
def _run(idx, tok_emb, pos_emb):
    k = pl.kernel(
        _body,
        out_type=jax.ShapeDtypeStruct((B * T, D), jnp.float32),
        mesh=plsc.VectorSubcoreMesh(core_axis_name="c", subcore_axis_name="s"),
        scratch_types=[
            pltpu.VMEM((B, T_PER_W), jnp.int32),
            pltpu.VMEM((2, C, D), jnp.float32),
            pltpu.VMEM((2, B, C, D), jnp.float32),
            pltpu.SemaphoreType.DMA,
            pltpu.SemaphoreType.DMA,
            pltpu.SemaphoreType.DMA,
        ],
    )
    return k(idx, tok_emb, pos_emb)


def kernel(idx, tok_emb, pos_emb):
    out = _run(idx.astype(jnp.int32), tok_emb, pos_emb)
    return out.reshape(B, T, D)
